# baseline (speedup 1.0000x reference)
import functools

import jax
import jax.numpy as jnp
from jax import lax
from jax.experimental import pallas as pl
from jax.experimental.pallas import tpu as pltpu
from jax.experimental.pallas import tpu_sc as plsc

B = 64
H = 512
W = 512
N = H * W


NGROUP = 4
SPG = (H // 8) // NGROUP


def _argmax_body(*refs):
    cls_refs = refs[:NGROUP]
    idx_ref = refs[NGROUP]
    groups = []
    for g in range(NGROUP):
        s0 = g * SPG
        ref = cls_refs[g]
        acc_v = ref[0, 0, pl.ds(0, 8), :]
        acc_t = jnp.full((8, W), s0, jnp.int32)
        for s in range(1, SPG):
            strip = ref[0, 0, pl.ds(s * 8, 8), :]
            cmp = strip > acc_v
            acc_v = jnp.where(cmp, strip, acc_v)
            acc_t = jnp.where(
                cmp, jnp.full((8, W), s0 + s, jnp.int32), acc_t)
        groups.append((acc_v, acc_t))
    while len(groups) > 1:
        nxt = []
        for (v1, t1), (v2, t2) in zip(groups[0::2], groups[1::2]):
            take2 = jnp.logical_or(v2 > v1,
                                   jnp.logical_and(v2 == v1, t2 < t1))
            nxt.append((jnp.where(take2, v2, v1), jnp.where(take2, t2, t1)))
        groups = nxt
    acc_v, acc_t = groups[0]
    m = jnp.max(acc_v)
    sub = lax.broadcasted_iota(jnp.int32, (8, W), 0)
    lane = lax.broadcasted_iota(jnp.int32, (8, W), 1)
    flat = (acc_t * 8 + sub) * W + lane
    idx = jnp.min(jnp.where(acc_v == m, flat, jnp.int32(N)))
    idx_ref[0, 0, 0] = idx


def _argmax_call(cls_input):
    hq = H // NGROUP

    def mk_map(g):
        return lambda i: (i, 0, g, 0)

    return pl.pallas_call(
        _argmax_body,
        grid=(B,),
        in_specs=[
            pl.BlockSpec((1, 1, hq, W), mk_map(g)) for g in range(NGROUP)
        ],
        out_specs=pl.BlockSpec(
            (1, 1, 1), lambda i: (i, 0, 0), memory_space=pltpu.SMEM
        ),
        out_shape=jax.ShapeDtypeStruct((B, 1, 1), jnp.int32),
    )(*([cls_input] * NGROUP))


def _stage_body(idx_ref, loc_any, rows_any, sem):
    copies = []
    for i in range(B):
        r = idx_ref[i, 0, 0] >> 9
        c0 = pltpu.make_async_copy(
            loc_any.at[i, 0, r], rows_any.at[pl.ds(i * W, W)], sem)
        c1 = pltpu.make_async_copy(
            loc_any.at[i, 1, r], rows_any.at[pl.ds((B + i) * W, W)], sem)
        c0.start()
        c1.start()
        copies.append(c0)
        copies.append(c1)
    for c in copies:
        c.wait()


def _stage_call(idx3, loc_input):
    return pl.pallas_call(
        _stage_body,
        in_specs=[
            pl.BlockSpec(memory_space=pltpu.SMEM),
            pl.BlockSpec(memory_space=pltpu.MemorySpace.HBM),
        ],
        out_specs=pl.BlockSpec(memory_space=pltpu.MemorySpace.HBM),
        out_shape=jax.ShapeDtypeStruct((2 * B * W,), jnp.float32),
        scratch_shapes=[pltpu.SemaphoreType.DMA],
    )(idx3, loc_input)


def _sc_loss_body(idx_hbm, cr_hbm, rows_hbm, out_hbm, idx_v, cr_v, off_v,
                  vals_v, out_v, sem):
    cid = lax.axis_index("c")
    sid = lax.axis_index("s")

    @pl.when(jnp.logical_and(cid == 0, sid == 0))
    def _():
        pltpu.sync_copy(idx_hbm, idx_v)
        pltpu.sync_copy(cr_hbm, cr_v)
        for k in range(4):
            idx_k = idx_v[pl.ds(16 * k, 16)]
            cc = idx_k & 511
            slot = lax.iota(jnp.int32, 16) + jnp.int32(16 * k)
            off_v[pl.ds(32 * k, 16)] = slot * jnp.int32(W) + cc
            off_v[pl.ds(32 * k + 16, 16)] = (
                (slot + jnp.int32(B)) * jnp.int32(W) + cc)
        pltpu.async_copy(rows_hbm.at[off_v], vals_v, sem).wait()
        acc = jnp.zeros((16,), jnp.float32)
        for k in range(4):
            idx_k = idx_v[pl.ds(16 * k, 16)]
            r = (idx_k >> 9).astype(jnp.float32)
            cc = (idx_k & 511).astype(jnp.float32)
            cr0 = cr_v[pl.ds(16 * k, 16)]
            cr1 = cr_v[pl.ds(64 + 16 * k, 16)]
            bias0 = cr0 * 511.0 - r
            bias1 = cr1 * 511.0 - cc
            v0 = vals_v[pl.ds(32 * k, 16)]
            v1 = vals_v[pl.ds(32 * k + 16, 16)]
            for v, bias in ((v0, bias0), (v1, bias1)):
                d = v - bias
                ad = jnp.abs(d)
                acc = acc + jnp.where(ad < 1.0, 0.5 * d * d, ad - 0.5)
        total = acc[0]
        for i in range(1, 16):
            total = total + acc[i]
        out_v[...] = jnp.broadcast_to(total * (1.0 / 128.0), (16,))
        pltpu.sync_copy(out_v, out_hbm)


@functools.partial(
    pl.kernel,
    out_type=jax.ShapeDtypeStruct((16,), jnp.float32),
    mesh=plsc.VectorSubcoreMesh(core_axis_name="c", subcore_axis_name="s"),
    scratch_types=[
        pltpu.VMEM((B,), jnp.int32),
        pltpu.VMEM((2 * B,), jnp.float32),
        pltpu.VMEM((2 * B,), jnp.int32),
        pltpu.VMEM((2 * B,), jnp.float32),
        pltpu.VMEM((16,), jnp.float32),
        pltpu.SemaphoreType.DMA,
    ],
)
def _sc_loss_call(idx_hbm, cr_hbm, rows_hbm, out_hbm, idx_v, cr_v, off_v,
                  vals_v, out_v, sem):
    _sc_loss_body(idx_hbm, cr_hbm, rows_hbm, out_hbm, idx_v, cr_v, off_v,
                  vals_v, out_v, sem)


def kernel(cls_input, loc_input, center_rate):
    idx3 = _argmax_call(cls_input)
    rows = _stage_call(idx3, loc_input)
    out = _sc_loss_call(idx3.reshape(B), center_rate.reshape(2 * B), rows)
    return out[0]


# speedup vs baseline: 1.2498x; 1.2498x over previous
import functools

import jax
import jax.numpy as jnp
from jax import lax
from jax.experimental import pallas as pl
from jax.experimental.pallas import tpu as pltpu
from jax.experimental.pallas import tpu_sc as plsc

B = 64
H = 512
W = 512
N = H * W


NGROUP = 4
SPG = (H // 8) // NGROUP


IPS = 2


def _argmax_body(*refs):
    cls_refs = refs[:NGROUP]
    idx_ref = refs[NGROUP]
    for img in range(IPS):
        groups = []
        for g in range(NGROUP):
            s0 = g * SPG
            ref = cls_refs[g]
            acc_v = ref[img, 0, pl.ds(0, 8), :]
            acc_t = jnp.full((8, W), s0, jnp.int32)
            for s in range(1, SPG):
                strip = ref[img, 0, pl.ds(s * 8, 8), :]
                cmp = strip > acc_v
                acc_v = jnp.where(cmp, strip, acc_v)
                acc_t = jnp.where(
                    cmp, jnp.full((8, W), s0 + s, jnp.int32), acc_t)
            groups.append((acc_v, acc_t))
        while len(groups) > 1:
            nxt = []
            for (v1, t1), (v2, t2) in zip(groups[0::2], groups[1::2]):
                take2 = jnp.logical_or(
                    v2 > v1, jnp.logical_and(v2 == v1, t2 < t1))
                nxt.append(
                    (jnp.where(take2, v2, v1), jnp.where(take2, t2, t1)))
            groups = nxt
        acc_v, acc_t = groups[0]
        m = jnp.max(acc_v)
        sub = lax.broadcasted_iota(jnp.int32, (8, W), 0)
        lane = lax.broadcasted_iota(jnp.int32, (8, W), 1)
        flat = (acc_t * 8 + sub) * W + lane
        idx = jnp.min(jnp.where(acc_v == m, flat, jnp.int32(N)))
        idx_ref[img, 0, 0] = idx


def _argmax_call(cls_input):
    hq = H // NGROUP

    def mk_map(g):
        return lambda i: (i, 0, g, 0)

    return pl.pallas_call(
        _argmax_body,
        grid=(B // IPS,),
        in_specs=[
            pl.BlockSpec((IPS, 1, hq, W), mk_map(g)) for g in range(NGROUP)
        ],
        out_specs=pl.BlockSpec(
            (IPS, 1, 1), lambda i: (i, 0, 0), memory_space=pltpu.SMEM
        ),
        out_shape=jax.ShapeDtypeStruct((B, 1, 1), jnp.int32),
    )(*([cls_input] * NGROUP))


def _stage_body(idx_ref, loc_any, rows_any, sem):
    copies = []
    for i in range(B):
        r = idx_ref[i, 0, 0] >> 9
        c0 = pltpu.make_async_copy(
            loc_any.at[i, 0, r], rows_any.at[pl.ds(i * W, W)], sem)
        c1 = pltpu.make_async_copy(
            loc_any.at[i, 1, r], rows_any.at[pl.ds((B + i) * W, W)], sem)
        c0.start()
        c1.start()
        copies.append(c0)
        copies.append(c1)
    for c in copies:
        c.wait()


def _stage_call(idx3, loc_input):
    return pl.pallas_call(
        _stage_body,
        in_specs=[
            pl.BlockSpec(memory_space=pltpu.SMEM),
            pl.BlockSpec(memory_space=pltpu.MemorySpace.HBM),
        ],
        out_specs=pl.BlockSpec(memory_space=pltpu.MemorySpace.HBM),
        out_shape=jax.ShapeDtypeStruct((2 * B * W,), jnp.float32),
        scratch_shapes=[pltpu.SemaphoreType.DMA],
    )(idx3, loc_input)


def _sc_loss_body(idx_hbm, cr_hbm, rows_hbm, out_hbm, idx_v, cr_v, off_v,
                  vals_v, out_v, sem):
    cid = lax.axis_index("c")
    sid = lax.axis_index("s")

    @pl.when(jnp.logical_and(cid == 0, sid == 0))
    def _():
        pltpu.sync_copy(idx_hbm, idx_v)
        pltpu.sync_copy(cr_hbm, cr_v)
        for k in range(4):
            idx_k = idx_v[pl.ds(16 * k, 16)]
            cc = idx_k & 511
            slot = lax.iota(jnp.int32, 16) + jnp.int32(16 * k)
            off_v[pl.ds(32 * k, 16)] = slot * jnp.int32(W) + cc
            off_v[pl.ds(32 * k + 16, 16)] = (
                (slot + jnp.int32(B)) * jnp.int32(W) + cc)
        pltpu.async_copy(rows_hbm.at[off_v], vals_v, sem).wait()
        acc = jnp.zeros((16,), jnp.float32)
        for k in range(4):
            idx_k = idx_v[pl.ds(16 * k, 16)]
            r = (idx_k >> 9).astype(jnp.float32)
            cc = (idx_k & 511).astype(jnp.float32)
            cr0 = cr_v[pl.ds(16 * k, 16)]
            cr1 = cr_v[pl.ds(64 + 16 * k, 16)]
            bias0 = cr0 * 511.0 - r
            bias1 = cr1 * 511.0 - cc
            v0 = vals_v[pl.ds(32 * k, 16)]
            v1 = vals_v[pl.ds(32 * k + 16, 16)]
            for v, bias in ((v0, bias0), (v1, bias1)):
                d = v - bias
                ad = jnp.abs(d)
                acc = acc + jnp.where(ad < 1.0, 0.5 * d * d, ad - 0.5)
        total = acc[0]
        for i in range(1, 16):
            total = total + acc[i]
        out_v[...] = jnp.broadcast_to(total * (1.0 / 128.0), (16,))
        pltpu.sync_copy(out_v, out_hbm)


@functools.partial(
    pl.kernel,
    out_type=jax.ShapeDtypeStruct((16,), jnp.float32),
    mesh=plsc.VectorSubcoreMesh(core_axis_name="c", subcore_axis_name="s"),
    scratch_types=[
        pltpu.VMEM((B,), jnp.int32),
        pltpu.VMEM((2 * B,), jnp.float32),
        pltpu.VMEM((2 * B,), jnp.int32),
        pltpu.VMEM((2 * B,), jnp.float32),
        pltpu.VMEM((16,), jnp.float32),
        pltpu.SemaphoreType.DMA,
    ],
)
def _sc_loss_call(idx_hbm, cr_hbm, rows_hbm, out_hbm, idx_v, cr_v, off_v,
                  vals_v, out_v, sem):
    _sc_loss_body(idx_hbm, cr_hbm, rows_hbm, out_hbm, idx_v, cr_v, off_v,
                  vals_v, out_v, sem)


def kernel(cls_input, loc_input, center_rate):
    idx3 = _argmax_call(cls_input)
    rows = _stage_call(idx3, loc_input)
    out = _sc_loss_call(idx3.reshape(B), center_rate.reshape(2 * B), rows)
    return out[0]


# speedup vs baseline: 1.4906x; 1.1926x over previous
import functools

import jax
import jax.numpy as jnp
from jax import lax
from jax.experimental import pallas as pl
from jax.experimental.pallas import tpu as pltpu
from jax.experimental.pallas import tpu_sc as plsc

B = 64
H = 512
W = 512
N = H * W


NGROUP = 4
SPG = (H // 8) // NGROUP


IPS = 4


def _argmax_body(*refs):
    cls_refs = refs[:NGROUP]
    idx_ref = refs[NGROUP]
    for img in range(IPS):
        groups = []
        for g in range(NGROUP):
            s0 = g * SPG
            ref = cls_refs[g]
            acc_v = ref[img, 0, pl.ds(0, 8), :]
            acc_t = jnp.full((8, W), s0, jnp.int32)
            for s in range(1, SPG):
                strip = ref[img, 0, pl.ds(s * 8, 8), :]
                cmp = strip > acc_v
                acc_v = jnp.where(cmp, strip, acc_v)
                acc_t = jnp.where(
                    cmp, jnp.full((8, W), s0 + s, jnp.int32), acc_t)
            groups.append((acc_v, acc_t))
        while len(groups) > 1:
            nxt = []
            for (v1, t1), (v2, t2) in zip(groups[0::2], groups[1::2]):
                take2 = jnp.logical_or(
                    v2 > v1, jnp.logical_and(v2 == v1, t2 < t1))
                nxt.append(
                    (jnp.where(take2, v2, v1), jnp.where(take2, t2, t1)))
            groups = nxt
        acc_v, acc_t = groups[0]
        m = jnp.max(acc_v)
        sub = lax.broadcasted_iota(jnp.int32, (8, W), 0)
        lane = lax.broadcasted_iota(jnp.int32, (8, W), 1)
        flat = (acc_t * 8 + sub) * W + lane
        idx = jnp.min(jnp.where(acc_v == m, flat, jnp.int32(N)))
        idx_ref[img, 0, 0] = idx


def _argmax_call(cls_input):
    hq = H // NGROUP

    def mk_map(g):
        return lambda i: (i, 0, g, 0)

    return pl.pallas_call(
        _argmax_body,
        grid=(B // IPS,),
        in_specs=[
            pl.BlockSpec((IPS, 1, hq, W), mk_map(g)) for g in range(NGROUP)
        ],
        out_specs=pl.BlockSpec(
            (IPS, 1, 1), lambda i: (i, 0, 0), memory_space=pltpu.SMEM
        ),
        out_shape=jax.ShapeDtypeStruct((B, 1, 1), jnp.int32),
    )(*([cls_input] * NGROUP))


def _stage_body(idx_ref, loc_any, rows_any, sem):
    copies = []
    for i in range(B):
        r = idx_ref[i, 0, 0] >> 9
        c0 = pltpu.make_async_copy(
            loc_any.at[i, 0, r], rows_any.at[pl.ds(i * W, W)], sem)
        c1 = pltpu.make_async_copy(
            loc_any.at[i, 1, r], rows_any.at[pl.ds((B + i) * W, W)], sem)
        c0.start()
        c1.start()
        copies.append(c0)
        copies.append(c1)
    for c in copies:
        c.wait()


def _stage_call(idx3, loc_input):
    return pl.pallas_call(
        _stage_body,
        in_specs=[
            pl.BlockSpec(memory_space=pltpu.SMEM),
            pl.BlockSpec(memory_space=pltpu.MemorySpace.HBM),
        ],
        out_specs=pl.BlockSpec(memory_space=pltpu.MemorySpace.HBM),
        out_shape=jax.ShapeDtypeStruct((2 * B * W,), jnp.float32),
        scratch_shapes=[pltpu.SemaphoreType.DMA],
    )(idx3, loc_input)


def _sc_loss_body(idx_hbm, cr_hbm, rows_hbm, out_hbm, idx_v, cr_v, off_v,
                  vals_v, out_v, sem):
    cid = lax.axis_index("c")
    sid = lax.axis_index("s")

    @pl.when(jnp.logical_and(cid == 0, sid == 0))
    def _():
        pltpu.sync_copy(idx_hbm, idx_v)
        pltpu.sync_copy(cr_hbm, cr_v)
        for k in range(4):
            idx_k = idx_v[pl.ds(16 * k, 16)]
            cc = idx_k & 511
            slot = lax.iota(jnp.int32, 16) + jnp.int32(16 * k)
            off_v[pl.ds(32 * k, 16)] = slot * jnp.int32(W) + cc
            off_v[pl.ds(32 * k + 16, 16)] = (
                (slot + jnp.int32(B)) * jnp.int32(W) + cc)
        pltpu.async_copy(rows_hbm.at[off_v], vals_v, sem).wait()
        acc = jnp.zeros((16,), jnp.float32)
        for k in range(4):
            idx_k = idx_v[pl.ds(16 * k, 16)]
            r = (idx_k >> 9).astype(jnp.float32)
            cc = (idx_k & 511).astype(jnp.float32)
            cr0 = cr_v[pl.ds(16 * k, 16)]
            cr1 = cr_v[pl.ds(64 + 16 * k, 16)]
            bias0 = cr0 * 511.0 - r
            bias1 = cr1 * 511.0 - cc
            v0 = vals_v[pl.ds(32 * k, 16)]
            v1 = vals_v[pl.ds(32 * k + 16, 16)]
            for v, bias in ((v0, bias0), (v1, bias1)):
                d = v - bias
                ad = jnp.abs(d)
                acc = acc + jnp.where(ad < 1.0, 0.5 * d * d, ad - 0.5)
        total = acc[0]
        for i in range(1, 16):
            total = total + acc[i]
        out_v[...] = jnp.broadcast_to(total * (1.0 / 128.0), (16,))
        pltpu.sync_copy(out_v, out_hbm)


@functools.partial(
    pl.kernel,
    out_type=jax.ShapeDtypeStruct((16,), jnp.float32),
    mesh=plsc.VectorSubcoreMesh(core_axis_name="c", subcore_axis_name="s"),
    scratch_types=[
        pltpu.VMEM((B,), jnp.int32),
        pltpu.VMEM((2 * B,), jnp.float32),
        pltpu.VMEM((2 * B,), jnp.int32),
        pltpu.VMEM((2 * B,), jnp.float32),
        pltpu.VMEM((16,), jnp.float32),
        pltpu.SemaphoreType.DMA,
    ],
)
def _sc_loss_call(idx_hbm, cr_hbm, rows_hbm, out_hbm, idx_v, cr_v, off_v,
                  vals_v, out_v, sem):
    _sc_loss_body(idx_hbm, cr_hbm, rows_hbm, out_hbm, idx_v, cr_v, off_v,
                  vals_v, out_v, sem)


def kernel(cls_input, loc_input, center_rate):
    idx3 = _argmax_call(cls_input)
    rows = _stage_call(idx3, loc_input)
    out = _sc_loss_call(idx3.reshape(B), center_rate.reshape(2 * B), rows)
    return out[0]


# speedup vs baseline: 1.5500x; 1.0399x over previous
import functools

import jax
import jax.numpy as jnp
from jax import lax
from jax.experimental import pallas as pl
from jax.experimental.pallas import tpu as pltpu
from jax.experimental.pallas import tpu_sc as plsc

B = 64
H = 512
W = 512
N = H * W


NGROUP = 4
SPG = (H // 8) // NGROUP


IPS = 8


def _argmax_body(*refs):
    cls_refs = refs[:NGROUP]
    idx_ref = refs[NGROUP]
    for img in range(IPS):
        groups = []
        for g in range(NGROUP):
            s0 = g * SPG
            ref = cls_refs[g]
            acc_v = ref[img, 0, pl.ds(0, 8), :]
            acc_t = jnp.full((8, W), s0, jnp.int32)
            for s in range(1, SPG):
                strip = ref[img, 0, pl.ds(s * 8, 8), :]
                cmp = strip > acc_v
                acc_v = jnp.where(cmp, strip, acc_v)
                acc_t = jnp.where(
                    cmp, jnp.full((8, W), s0 + s, jnp.int32), acc_t)
            groups.append((acc_v, acc_t))
        while len(groups) > 1:
            nxt = []
            for (v1, t1), (v2, t2) in zip(groups[0::2], groups[1::2]):
                take2 = jnp.logical_or(
                    v2 > v1, jnp.logical_and(v2 == v1, t2 < t1))
                nxt.append(
                    (jnp.where(take2, v2, v1), jnp.where(take2, t2, t1)))
            groups = nxt
        acc_v, acc_t = groups[0]
        m = jnp.max(acc_v)
        sub = lax.broadcasted_iota(jnp.int32, (8, W), 0)
        lane = lax.broadcasted_iota(jnp.int32, (8, W), 1)
        flat = (acc_t * 8 + sub) * W + lane
        idx = jnp.min(jnp.where(acc_v == m, flat, jnp.int32(N)))
        idx_ref[img, 0, 0] = idx


def _argmax_call(cls_input):
    hq = H // NGROUP

    def mk_map(g):
        return lambda i: (i, 0, g, 0)

    return pl.pallas_call(
        _argmax_body,
        grid=(B // IPS,),
        in_specs=[
            pl.BlockSpec((IPS, 1, hq, W), mk_map(g)) for g in range(NGROUP)
        ],
        out_specs=pl.BlockSpec(
            (IPS, 1, 1), lambda i: (i, 0, 0), memory_space=pltpu.SMEM
        ),
        out_shape=jax.ShapeDtypeStruct((B, 1, 1), jnp.int32),
    )(*([cls_input] * NGROUP))


def _stage_body(idx_ref, loc_any, rows_any, sem):
    copies = []
    for i in range(B):
        r = idx_ref[i, 0, 0] >> 9
        c0 = pltpu.make_async_copy(
            loc_any.at[i, 0, r], rows_any.at[pl.ds(i * W, W)], sem)
        c1 = pltpu.make_async_copy(
            loc_any.at[i, 1, r], rows_any.at[pl.ds((B + i) * W, W)], sem)
        c0.start()
        c1.start()
        copies.append(c0)
        copies.append(c1)
    for c in copies:
        c.wait()


def _stage_call(idx3, loc_input):
    return pl.pallas_call(
        _stage_body,
        in_specs=[
            pl.BlockSpec(memory_space=pltpu.SMEM),
            pl.BlockSpec(memory_space=pltpu.MemorySpace.HBM),
        ],
        out_specs=pl.BlockSpec(memory_space=pltpu.MemorySpace.HBM),
        out_shape=jax.ShapeDtypeStruct((2 * B * W,), jnp.float32),
        scratch_shapes=[pltpu.SemaphoreType.DMA],
    )(idx3, loc_input)


def _sc_loss_body(idx_hbm, cr_hbm, rows_hbm, out_hbm, idx_v, cr_v, off_v,
                  vals_v, out_v, sem):
    cid = lax.axis_index("c")
    sid = lax.axis_index("s")

    @pl.when(jnp.logical_and(cid == 0, sid == 0))
    def _():
        pltpu.sync_copy(idx_hbm, idx_v)
        pltpu.sync_copy(cr_hbm, cr_v)
        for k in range(4):
            idx_k = idx_v[pl.ds(16 * k, 16)]
            cc = idx_k & 511
            slot = lax.iota(jnp.int32, 16) + jnp.int32(16 * k)
            off_v[pl.ds(32 * k, 16)] = slot * jnp.int32(W) + cc
            off_v[pl.ds(32 * k + 16, 16)] = (
                (slot + jnp.int32(B)) * jnp.int32(W) + cc)
        pltpu.async_copy(rows_hbm.at[off_v], vals_v, sem).wait()
        acc = jnp.zeros((16,), jnp.float32)
        for k in range(4):
            idx_k = idx_v[pl.ds(16 * k, 16)]
            r = (idx_k >> 9).astype(jnp.float32)
            cc = (idx_k & 511).astype(jnp.float32)
            cr0 = cr_v[pl.ds(16 * k, 16)]
            cr1 = cr_v[pl.ds(64 + 16 * k, 16)]
            bias0 = cr0 * 511.0 - r
            bias1 = cr1 * 511.0 - cc
            v0 = vals_v[pl.ds(32 * k, 16)]
            v1 = vals_v[pl.ds(32 * k + 16, 16)]
            for v, bias in ((v0, bias0), (v1, bias1)):
                d = v - bias
                ad = jnp.abs(d)
                acc = acc + jnp.where(ad < 1.0, 0.5 * d * d, ad - 0.5)
        total = acc[0]
        for i in range(1, 16):
            total = total + acc[i]
        out_v[...] = jnp.broadcast_to(total * (1.0 / 128.0), (16,))
        pltpu.sync_copy(out_v, out_hbm)


@functools.partial(
    pl.kernel,
    out_type=jax.ShapeDtypeStruct((16,), jnp.float32),
    mesh=plsc.VectorSubcoreMesh(core_axis_name="c", subcore_axis_name="s"),
    scratch_types=[
        pltpu.VMEM((B,), jnp.int32),
        pltpu.VMEM((2 * B,), jnp.float32),
        pltpu.VMEM((2 * B,), jnp.int32),
        pltpu.VMEM((2 * B,), jnp.float32),
        pltpu.VMEM((16,), jnp.float32),
        pltpu.SemaphoreType.DMA,
    ],
)
def _sc_loss_call(idx_hbm, cr_hbm, rows_hbm, out_hbm, idx_v, cr_v, off_v,
                  vals_v, out_v, sem):
    _sc_loss_body(idx_hbm, cr_hbm, rows_hbm, out_hbm, idx_v, cr_v, off_v,
                  vals_v, out_v, sem)


def kernel(cls_input, loc_input, center_rate):
    idx3 = _argmax_call(cls_input)
    rows = _stage_call(idx3, loc_input)
    out = _sc_loss_call(idx3.reshape(B), center_rate.reshape(2 * B), rows)
    return out[0]


# speedup vs baseline: 1.5594x; 1.0060x over previous
import functools

import jax
import jax.numpy as jnp
from jax import lax
from jax.experimental import pallas as pl
from jax.experimental.pallas import tpu as pltpu
from jax.experimental.pallas import tpu_sc as plsc

B = 64
H = 512
W = 512
N = H * W


NGROUP = 4
SPG = (H // 8) // NGROUP


IPS = 16


def _argmax_body(*refs):
    cls_refs = refs[:NGROUP]
    idx_ref = refs[NGROUP]
    for img in range(IPS):
        groups = []
        for g in range(NGROUP):
            s0 = g * SPG
            ref = cls_refs[g]
            acc_v = ref[img, 0, pl.ds(0, 8), :]
            acc_t = jnp.full((8, W), s0, jnp.int32)
            for s in range(1, SPG):
                strip = ref[img, 0, pl.ds(s * 8, 8), :]
                cmp = strip > acc_v
                acc_v = jnp.where(cmp, strip, acc_v)
                acc_t = jnp.where(
                    cmp, jnp.full((8, W), s0 + s, jnp.int32), acc_t)
            groups.append((acc_v, acc_t))
        while len(groups) > 1:
            nxt = []
            for (v1, t1), (v2, t2) in zip(groups[0::2], groups[1::2]):
                take2 = jnp.logical_or(
                    v2 > v1, jnp.logical_and(v2 == v1, t2 < t1))
                nxt.append(
                    (jnp.where(take2, v2, v1), jnp.where(take2, t2, t1)))
            groups = nxt
        acc_v, acc_t = groups[0]
        m = jnp.max(acc_v)
        sub = lax.broadcasted_iota(jnp.int32, (8, W), 0)
        lane = lax.broadcasted_iota(jnp.int32, (8, W), 1)
        flat = (acc_t * 8 + sub) * W + lane
        idx = jnp.min(jnp.where(acc_v == m, flat, jnp.int32(N)))
        idx_ref[img, 0, 0] = idx


def _argmax_call(cls_input):
    hq = H // NGROUP

    def mk_map(g):
        return lambda i: (i, 0, g, 0)

    return pl.pallas_call(
        _argmax_body,
        grid=(B // IPS,),
        in_specs=[
            pl.BlockSpec((IPS, 1, hq, W), mk_map(g)) for g in range(NGROUP)
        ],
        out_specs=pl.BlockSpec(
            (IPS, 1, 1), lambda i: (i, 0, 0), memory_space=pltpu.SMEM
        ),
        out_shape=jax.ShapeDtypeStruct((B, 1, 1), jnp.int32),
    )(*([cls_input] * NGROUP))


def _stage_body(idx_ref, loc_any, rows_any, sem):
    copies = []
    for i in range(B):
        r = idx_ref[i, 0, 0] >> 9
        c0 = pltpu.make_async_copy(
            loc_any.at[i, 0, r], rows_any.at[pl.ds(i * W, W)], sem)
        c1 = pltpu.make_async_copy(
            loc_any.at[i, 1, r], rows_any.at[pl.ds((B + i) * W, W)], sem)
        c0.start()
        c1.start()
        copies.append(c0)
        copies.append(c1)
    for c in copies:
        c.wait()


def _stage_call(idx3, loc_input):
    return pl.pallas_call(
        _stage_body,
        in_specs=[
            pl.BlockSpec(memory_space=pltpu.SMEM),
            pl.BlockSpec(memory_space=pltpu.MemorySpace.HBM),
        ],
        out_specs=pl.BlockSpec(memory_space=pltpu.MemorySpace.HBM),
        out_shape=jax.ShapeDtypeStruct((2 * B * W,), jnp.float32),
        scratch_shapes=[pltpu.SemaphoreType.DMA],
    )(idx3, loc_input)


def _sc_loss_body(idx_hbm, cr_hbm, rows_hbm, out_hbm, idx_v, cr_v, off_v,
                  vals_v, out_v, sem):
    cid = lax.axis_index("c")
    sid = lax.axis_index("s")

    @pl.when(jnp.logical_and(cid == 0, sid == 0))
    def _():
        pltpu.sync_copy(idx_hbm, idx_v)
        pltpu.sync_copy(cr_hbm, cr_v)
        for k in range(4):
            idx_k = idx_v[pl.ds(16 * k, 16)]
            cc = idx_k & 511
            slot = lax.iota(jnp.int32, 16) + jnp.int32(16 * k)
            off_v[pl.ds(32 * k, 16)] = slot * jnp.int32(W) + cc
            off_v[pl.ds(32 * k + 16, 16)] = (
                (slot + jnp.int32(B)) * jnp.int32(W) + cc)
        pltpu.async_copy(rows_hbm.at[off_v], vals_v, sem).wait()
        acc = jnp.zeros((16,), jnp.float32)
        for k in range(4):
            idx_k = idx_v[pl.ds(16 * k, 16)]
            r = (idx_k >> 9).astype(jnp.float32)
            cc = (idx_k & 511).astype(jnp.float32)
            cr0 = cr_v[pl.ds(16 * k, 16)]
            cr1 = cr_v[pl.ds(64 + 16 * k, 16)]
            bias0 = cr0 * 511.0 - r
            bias1 = cr1 * 511.0 - cc
            v0 = vals_v[pl.ds(32 * k, 16)]
            v1 = vals_v[pl.ds(32 * k + 16, 16)]
            for v, bias in ((v0, bias0), (v1, bias1)):
                d = v - bias
                ad = jnp.abs(d)
                acc = acc + jnp.where(ad < 1.0, 0.5 * d * d, ad - 0.5)
        total = acc[0]
        for i in range(1, 16):
            total = total + acc[i]
        out_v[...] = jnp.broadcast_to(total * (1.0 / 128.0), (16,))
        pltpu.sync_copy(out_v, out_hbm)


@functools.partial(
    pl.kernel,
    out_type=jax.ShapeDtypeStruct((16,), jnp.float32),
    mesh=plsc.VectorSubcoreMesh(core_axis_name="c", subcore_axis_name="s"),
    scratch_types=[
        pltpu.VMEM((B,), jnp.int32),
        pltpu.VMEM((2 * B,), jnp.float32),
        pltpu.VMEM((2 * B,), jnp.int32),
        pltpu.VMEM((2 * B,), jnp.float32),
        pltpu.VMEM((16,), jnp.float32),
        pltpu.SemaphoreType.DMA,
    ],
)
def _sc_loss_call(idx_hbm, cr_hbm, rows_hbm, out_hbm, idx_v, cr_v, off_v,
                  vals_v, out_v, sem):
    _sc_loss_body(idx_hbm, cr_hbm, rows_hbm, out_hbm, idx_v, cr_v, off_v,
                  vals_v, out_v, sem)


def kernel(cls_input, loc_input, center_rate):
    idx3 = _argmax_call(cls_input)
    rows = _stage_call(idx3, loc_input)
    out = _sc_loss_call(idx3.reshape(B), center_rate.reshape(2 * B), rows)
    return out[0]
